# Initial kernel scaffold; baseline (speedup 1.0000x reference)
import jax
import jax.numpy as jnp
from jax.experimental import pallas as pl


def kernel(x_user, x_item, edge_clicks, edge_clicked_by, neg_edge_clicks, W1_clicks, W1_clicked_by, b1, W2_clicks, W2_clicked_by, b2, Wp, bp):
    raise NotImplementedError("write your pallas kernel here")


# speedup vs baseline: 3.9103x; 3.9103x over previous
import jax
import jax.numpy as jnp
from jax import lax
from jax.experimental import pallas as pl
from jax.experimental.pallas import tpu as pltpu
from jax.experimental.pallas import tpu_sc as plsc

N = 10000
E = 160000
DF = 128
NPAD = 10240
NSUB = 16
NCORE = 2
CH = 100
NCHUNK = 100
EPT = CH * NCHUNK
SLAB = NPAD // NSUB

_mesh = plsc.VectorSubcoreMesh(core_axis_name="c", subcore_axis_name="s",
                               num_cores=NCORE, num_subcores=NSUB)


def _deg_body(dst_c, dst_cb, zeros, ones, deg_i, deg_u,
              dst_v, ones_v, acc):
    cid = lax.axis_index("c")
    sid = lax.axis_index("s")
    slab = pl.ds(sid * SLAB, SLAB)
    pltpu.sync_copy(ones, ones_v)
    pltpu.sync_copy(zeros.at[slab], acc.at[slab])
    plsc.subcore_barrier()

    def run(dsts):
        pltpu.sync_copy(dsts.at[sid], dst_v)

        def deg_chunk(j, carry):
            pltpu.sync_copy(ones_v, acc.at[dst_v.at[j]], add=True)
            return carry

        lax.fori_loop(0, NCHUNK, deg_chunk, 0)

    pl.when(cid == 0)(lambda: run(dst_c))
    pl.when(cid == 1)(lambda: run(dst_cb))
    plsc.subcore_barrier()
    pl.when(cid == 0)(lambda: pltpu.sync_copy(acc.at[slab], deg_i.at[slab]))
    pl.when(cid == 1)(lambda: pltpu.sync_copy(acc.at[slab], deg_u.at[slab]))


_deg = pl.kernel(
    _deg_body,
    out_type=(jax.ShapeDtypeStruct((NPAD, DF), jnp.float32),
              jax.ShapeDtypeStruct((NPAD, DF), jnp.float32)),
    mesh=_mesh,
    scratch_types=(
        pltpu.VMEM((NCHUNK, CH), jnp.int32),
        pltpu.VMEM((CH, DF), jnp.float32),
        pltpu.VMEM_SHARED((NPAD, DF), jnp.float32),
    ),
)


def _agg_body(mat_c, mat_cb, src_c, dst_c, src_cb, dst_cb, zeros,
              out_i, out_u, src_v, dst_v, rows_v, sem, acc):
    cid = lax.axis_index("c")
    sid = lax.axis_index("s")
    slab = pl.ds(sid * SLAB, SLAB)
    pltpu.sync_copy(zeros.at[slab], acc.at[slab])
    plsc.subcore_barrier()

    def run(mat, srcs, dsts):
        pltpu.sync_copy(srcs.at[sid], src_v)
        pltpu.sync_copy(dsts.at[sid], dst_v)

        def chunk(j, carry):
            pltpu.async_copy(mat.at[src_v.at[j]], rows_v, sem).wait()
            pltpu.sync_copy(rows_v, acc.at[dst_v.at[j]], add=True)
            return carry

        lax.fori_loop(0, NCHUNK, chunk, 0)

    pl.when(cid == 0)(lambda: run(mat_c, src_c, dst_c))
    pl.when(cid == 1)(lambda: run(mat_cb, src_cb, dst_cb))
    plsc.subcore_barrier()
    pl.when(cid == 0)(lambda: pltpu.sync_copy(acc.at[slab], out_i.at[slab]))
    pl.when(cid == 1)(lambda: pltpu.sync_copy(acc.at[slab], out_u.at[slab]))


_agg = pl.kernel(
    _agg_body,
    out_type=(jax.ShapeDtypeStruct((NPAD, DF), jnp.float32),
              jax.ShapeDtypeStruct((NPAD, DF), jnp.float32)),
    mesh=_mesh,
    scratch_types=(
        pltpu.VMEM((NCHUNK, CH), jnp.int32),
        pltpu.VMEM((NCHUNK, CH), jnp.int32),
        pltpu.VMEM((CH, DF), jnp.float32),
        pltpu.SemaphoreType.DMA,
        pltpu.VMEM_SHARED((NPAD, DF), jnp.float32),
    ),
)


PCH = 80
PNC = 125


def _pred_body(qa, qb, pos_e, neg_e, out_pos, out_neg,
               es_v, ed_v, a_v, b_v, o_v, sem):
    cid = lax.axis_index("c")
    sid = lax.axis_index("s")

    def run(edges, out):
        pltpu.sync_copy(edges.at[0, sid], es_v)
        pltpu.sync_copy(edges.at[1, sid], ed_v)

        def chunk(j, carry):
            pltpu.async_copy(qa.at[es_v.at[j]], a_v, sem).wait()
            pltpu.async_copy(qb.at[ed_v.at[j]], b_v, sem).wait()

            def row(k, c2):
                o_v[k, pl.ds(0, 16)] = (a_v[k, pl.ds(0, 16)]
                                        + b_v[k, pl.ds(0, 16)])
                return c2

            lax.fori_loop(0, PCH, row, 0)
            pltpu.sync_copy(o_v, out.at[pl.ds(sid * EPT + j * PCH, PCH)])
            return carry

        lax.fori_loop(0, PNC, chunk, 0)

    pl.when(cid == 0)(lambda: run(pos_e, out_pos))
    pl.when(cid == 1)(lambda: run(neg_e, out_neg))


_pred = pl.kernel(
    _pred_body,
    out_type=(jax.ShapeDtypeStruct((E, 16), jnp.float32),
              jax.ShapeDtypeStruct((E, 16), jnp.float32)),
    mesh=_mesh,
    scratch_types=(
        pltpu.VMEM((PNC, PCH), jnp.int32),
        pltpu.VMEM((PNC, PCH), jnp.int32),
        pltpu.VMEM((PCH, DF), jnp.float32),
        pltpu.VMEM((PCH, DF), jnp.float32),
        pltpu.VMEM((PCH, 16), jnp.float32),
        pltpu.SemaphoreType.DMA,
    ),
)


_TCM = 2000


def _tc_pre(xu, xi, w1c, w1cb):
    def body(xu_ref, xi_ref, wc_ref, wcb_ref, oc_ref, ocb_ref):
        oc_ref[...] = jnp.dot(xu_ref[...], wc_ref[...],
                              preferred_element_type=jnp.float32)
        ocb_ref[...] = jnp.dot(xi_ref[...], wcb_ref[...],
                               preferred_element_type=jnp.float32)

    return pl.pallas_call(
        body,
        grid=(N // _TCM,),
        in_specs=[pl.BlockSpec((_TCM, DF), lambda i: (i, 0)),
                  pl.BlockSpec((_TCM, DF), lambda i: (i, 0)),
                  pl.BlockSpec((DF, DF), lambda i: (0, 0)),
                  pl.BlockSpec((DF, DF), lambda i: (0, 0))],
        out_specs=[pl.BlockSpec((_TCM, DF), lambda i: (i, 0)),
                   pl.BlockSpec((_TCM, DF), lambda i: (i, 0))],
        out_shape=[jax.ShapeDtypeStruct((N, DF), jnp.float32),
                   jax.ShapeDtypeStruct((N, DF), jnp.float32)],
    )(xu, xi, w1c, w1cb)


def _tc_mid(s_i, s_u, deg_i, deg_u, b1, w2c, w2cb):
    def body(si_ref, su_ref, di_ref, du_ref, b_ref, wc_ref, wcb_ref,
             oc_ref, ocb_ref):
        b = b_ref[...]
        di = jnp.maximum(di_ref[...], 1.0)
        du = jnp.maximum(du_ref[...], 1.0)
        hi = jnp.maximum(si_ref[...] / di + b, 0.0)
        hu = jnp.maximum(su_ref[...] / du + b, 0.0)
        oc_ref[...] = jnp.dot(hu, wc_ref[...], preferred_element_type=jnp.float32)
        ocb_ref[...] = jnp.dot(hi, wcb_ref[...], preferred_element_type=jnp.float32)

    return pl.pallas_call(
        body,
        grid=(N // _TCM,),
        in_specs=[pl.BlockSpec((_TCM, DF), lambda i: (i, 0)),
                  pl.BlockSpec((_TCM, DF), lambda i: (i, 0)),
                  pl.BlockSpec((_TCM, 1), lambda i: (i, 0)),
                  pl.BlockSpec((_TCM, 1), lambda i: (i, 0)),
                  pl.BlockSpec((1, DF), lambda i: (0, 0)),
                  pl.BlockSpec((DF, DF), lambda i: (0, 0)),
                  pl.BlockSpec((DF, DF), lambda i: (0, 0))],
        out_specs=[pl.BlockSpec((_TCM, DF), lambda i: (i, 0)),
                   pl.BlockSpec((_TCM, DF), lambda i: (i, 0))],
        out_shape=[jax.ShapeDtypeStruct((N, DF), jnp.float32),
                   jax.ShapeDtypeStruct((N, DF), jnp.float32)],
    )(s_i, s_u, deg_i, deg_u, b1, w2c, w2cb)


def _tc_fin(s2i, s2u, deg_i, deg_u, b2, wpu, wpi, bppad):
    def body(si_ref, su_ref, di_ref, du_ref, b_ref, wpu_ref, wpi_ref,
             bp_ref, oa_ref, ob_ref):
        b = b_ref[...]
        di = jnp.maximum(di_ref[...], 1.0)
        du = jnp.maximum(du_ref[...], 1.0)
        hi = si_ref[...] / di + b
        hu = su_ref[...] / du + b
        oa_ref[...] = jnp.dot(hu, wpu_ref[...],
                              preferred_element_type=jnp.float32)
        ob_ref[...] = jnp.dot(hi, wpi_ref[...],
                              preferred_element_type=jnp.float32) + bp_ref[...]

    return pl.pallas_call(
        body,
        grid=(N // _TCM,),
        in_specs=[pl.BlockSpec((_TCM, DF), lambda i: (i, 0)),
                  pl.BlockSpec((_TCM, DF), lambda i: (i, 0)),
                  pl.BlockSpec((_TCM, 1), lambda i: (i, 0)),
                  pl.BlockSpec((_TCM, 1), lambda i: (i, 0)),
                  pl.BlockSpec((1, DF), lambda i: (0, 0)),
                  pl.BlockSpec((DF, DF), lambda i: (0, 0)),
                  pl.BlockSpec((DF, DF), lambda i: (0, 0)),
                  pl.BlockSpec((1, DF), lambda i: (0, 0))],
        out_specs=[pl.BlockSpec((_TCM, DF), lambda i: (i, 0)),
                   pl.BlockSpec((_TCM, DF), lambda i: (i, 0))],
        out_shape=[jax.ShapeDtypeStruct((N, DF), jnp.float32),
                   jax.ShapeDtypeStruct((N, DF), jnp.float32)],
    )(s2i, s2u, deg_i, deg_u, b2, wpu, wpi, bppad)


def _prep_edges(edges):
    src = edges[0].reshape(NSUB, NCHUNK, CH)
    dst = edges[1].reshape(NSUB, NCHUNK, CH)
    return src, dst


def kernel(x_user, x_item, edge_clicks, edge_clicked_by, neg_edge_clicks,
           W1_clicks, W1_clicked_by, b1, W2_clicks, W2_clicked_by, b2, Wp, bp):
    src_c, dst_c = _prep_edges(edge_clicks)
    src_cb, dst_cb = _prep_edges(edge_clicked_by)
    zeros = jnp.zeros((NPAD, DF), jnp.float32)
    ones = jnp.ones((CH, DF), jnp.float32)

    m1c, m1cb = _tc_pre(x_user, x_item, W1_clicks, W1_clicked_by)
    dgi, dgu = _deg(dst_c, dst_cb, zeros, ones)
    s1i, s1u = _agg(m1c, m1cb, src_c, dst_c, src_cb, dst_cb, zeros)
    deg_i = dgi[:, :1]
    deg_u = dgu[:, :1]
    m2c, m2cb = _tc_mid(s1i, s1u, deg_i, deg_u, b1.reshape(1, DF),
                        W2_clicks, W2_clicked_by)
    s2i, s2u = _agg(m2c, m2cb, src_c, dst_c, src_cb, dst_cb, zeros)

    wpu = jnp.pad(Wp[:DF], ((0, 0), (0, DF - 2)))
    wpi = jnp.pad(Wp[DF:], ((0, 0), (0, DF - 2)))
    bppad = jnp.pad(bp, (0, DF - 2)).reshape(1, DF)
    qa, qb = _tc_fin(s2i, s2u, deg_i, deg_u, b2.reshape(1, DF),
                     wpu, wpi, bppad)

    pos_e = edge_clicks.reshape(2, NSUB, PNC, PCH)
    neg_e = neg_edge_clicks.reshape(2, NSUB, PNC, PCH)
    pos_f, neg_f = _pred(qa, qb, pos_e, neg_e)
    return pos_f[:, :2], neg_f[:, :2]


# speedup vs baseline: 5.1402x; 1.3145x over previous
import jax
import jax.numpy as jnp
from jax import lax
from jax.experimental import pallas as pl
from jax.experimental.pallas import tpu as pltpu
from jax.experimental.pallas import tpu_sc as plsc

N = 10000
E = 160000
DF = 128
NPAD = 10240
NSUB = 16
NCORE = 2
CH = 125
NCHUNK = 80
EPT = CH * NCHUNK
SLAB = NPAD // NSUB

_mesh = plsc.VectorSubcoreMesh(core_axis_name="c", subcore_axis_name="s",
                               num_cores=NCORE, num_subcores=NSUB)


def _deg_body(dst_c, dst_cb, zeros, ones, deg_i, deg_u,
              dst_v, ones_v, acc):
    cid = lax.axis_index("c")
    sid = lax.axis_index("s")
    slab = pl.ds(sid * SLAB, SLAB)
    pltpu.sync_copy(ones, ones_v)
    pltpu.sync_copy(zeros.at[slab], acc.at[slab])
    plsc.subcore_barrier()

    def run(dsts):
        pltpu.sync_copy(dsts.at[sid], dst_v)

        def deg_chunk(j, carry):
            pltpu.sync_copy(ones_v, acc.at[dst_v.at[j]], add=True)
            return carry

        lax.fori_loop(0, NCHUNK, deg_chunk, 0)

    pl.when(cid == 0)(lambda: run(dst_c))
    pl.when(cid == 1)(lambda: run(dst_cb))
    plsc.subcore_barrier()
    pl.when(cid == 0)(lambda: pltpu.sync_copy(acc.at[slab], deg_i.at[slab]))
    pl.when(cid == 1)(lambda: pltpu.sync_copy(acc.at[slab], deg_u.at[slab]))


_deg = pl.kernel(
    _deg_body,
    out_type=(jax.ShapeDtypeStruct((NPAD, DF), jnp.float32),
              jax.ShapeDtypeStruct((NPAD, DF), jnp.float32)),
    mesh=_mesh,
    scratch_types=(
        pltpu.VMEM((NCHUNK, CH), jnp.int32),
        pltpu.VMEM((CH, DF), jnp.float32),
        pltpu.VMEM_SHARED((NPAD, DF), jnp.float32),
    ),
)


def _agg_body(mat_c, mat_cb, src_c, dst_c, src_cb, dst_cb, zeros,
              out_i, out_u, src_v, dst_v, rows, acc):
    cid = lax.axis_index("c")
    sid = lax.axis_index("s")
    slab = pl.ds(sid * SLAB, SLAB)
    pltpu.sync_copy(zeros.at[slab], acc.at[slab])
    plsc.subcore_barrier()

    def run(mat, srcs, dsts):
        pltpu.sync_copy(srcs.at[sid], src_v)
        pltpu.sync_copy(dsts.at[sid], dst_v)

        def chunk(j, carry):
            pltpu.sync_copy(mat.at[src_v.at[j]], rows)
            pltpu.sync_copy(rows, acc.at[dst_v.at[j]], add=True)
            return carry

        lax.fori_loop(0, NCHUNK, chunk, 0)

    pl.when(cid == 0)(lambda: run(mat_c, src_c, dst_c))
    pl.when(cid == 1)(lambda: run(mat_cb, src_cb, dst_cb))
    plsc.subcore_barrier()
    pl.when(cid == 0)(lambda: pltpu.sync_copy(acc.at[slab], out_i.at[slab]))
    pl.when(cid == 1)(lambda: pltpu.sync_copy(acc.at[slab], out_u.at[slab]))


_agg = pl.kernel(
    _agg_body,
    out_type=(jax.ShapeDtypeStruct((NPAD, DF), jnp.float32),
              jax.ShapeDtypeStruct((NPAD, DF), jnp.float32)),
    mesh=_mesh,
    scratch_types=(
        pltpu.VMEM((NCHUNK, CH), jnp.int32),
        pltpu.VMEM((NCHUNK, CH), jnp.int32),
        pltpu.VMEM((CH, DF), jnp.float32),
        pltpu.VMEM_SHARED((NPAD, DF), jnp.float32),
    ),
)


PCH = 80
PNC = 125


def _pred_body(qa, qb, pos_e, neg_e, out_pos, out_neg,
               es_v, ed_v, a0, b0, a1, b1, o_v,
               sa0, sb0, sa1, sb1):
    cid = lax.axis_index("c")
    sid = lax.axis_index("s")

    def run(edges, out):
        pltpu.sync_copy(edges.at[0, sid], es_v)
        pltpu.sync_copy(edges.at[1, sid], ed_v)

        def gather(j, av, bv, semav, sembv):
            pltpu.async_copy(qa.at[es_v.at[j]], av, semav)
            pltpu.async_copy(qb.at[ed_v.at[j]], bv, sembv)

        def drain(j, av, bv, semav, sembv):
            pltpu.make_async_copy(qa.at[es_v.at[j]], av, semav).wait()
            pltpu.make_async_copy(qb.at[ed_v.at[j]], bv, sembv).wait()

        def combine(j, av, bv):
            for k in range(PCH):
                o_v[k, pl.ds(0, 16)] = (av[k, pl.ds(0, 16)]
                                        + bv[k, pl.ds(0, 16)])
            pltpu.sync_copy(o_v, out.at[pl.ds(sid * EPT + j * PCH, PCH)])

        gather(0, a0, b0, sa0, sb0)

        def pair(jj, carry):
            j0 = jj * 2
            gather(j0 + 1, a1, b1, sa1, sb1)
            drain(j0, a0, b0, sa0, sb0)
            combine(j0, a0, b0)
            gather((j0 + 2) % PNC, a0, b0, sa0, sb0)
            drain(j0 + 1, a1, b1, sa1, sb1)
            combine(j0 + 1, a1, b1)
            return carry

        lax.fori_loop(0, PNC // 2, pair, 0)
        drain(PNC - 1, a0, b0, sa0, sb0)
        combine(PNC - 1, a0, b0)

    pl.when(cid == 0)(lambda: run(pos_e, out_pos))
    pl.when(cid == 1)(lambda: run(neg_e, out_neg))


_pred = pl.kernel(
    _pred_body,
    out_type=(jax.ShapeDtypeStruct((E, 16), jnp.float32),
              jax.ShapeDtypeStruct((E, 16), jnp.float32)),
    mesh=_mesh,
    scratch_types=(
        pltpu.VMEM((PNC, PCH), jnp.int32),
        pltpu.VMEM((PNC, PCH), jnp.int32),
        pltpu.VMEM((PCH, DF), jnp.float32),
        pltpu.VMEM((PCH, DF), jnp.float32),
        pltpu.VMEM((PCH, DF), jnp.float32),
        pltpu.VMEM((PCH, DF), jnp.float32),
        pltpu.VMEM((PCH, 16), jnp.float32),
        pltpu.SemaphoreType.DMA,
        pltpu.SemaphoreType.DMA,
        pltpu.SemaphoreType.DMA,
        pltpu.SemaphoreType.DMA,
    ),
)


_TCM = 2000


def _tc_pre(xu, xi, w1c, w1cb):
    def body(xu_ref, xi_ref, wc_ref, wcb_ref, oc_ref, ocb_ref):
        oc_ref[...] = jnp.dot(xu_ref[...], wc_ref[...],
                              preferred_element_type=jnp.float32)
        ocb_ref[...] = jnp.dot(xi_ref[...], wcb_ref[...],
                               preferred_element_type=jnp.float32)

    return pl.pallas_call(
        body,
        grid=(N // _TCM,),
        in_specs=[pl.BlockSpec((_TCM, DF), lambda i: (i, 0)),
                  pl.BlockSpec((_TCM, DF), lambda i: (i, 0)),
                  pl.BlockSpec((DF, DF), lambda i: (0, 0)),
                  pl.BlockSpec((DF, DF), lambda i: (0, 0))],
        out_specs=[pl.BlockSpec((_TCM, DF), lambda i: (i, 0)),
                   pl.BlockSpec((_TCM, DF), lambda i: (i, 0))],
        out_shape=[jax.ShapeDtypeStruct((N, DF), jnp.float32),
                   jax.ShapeDtypeStruct((N, DF), jnp.float32)],
    )(xu, xi, w1c, w1cb)


def _tc_mid(s_i, s_u, deg_i, deg_u, b1, w2c, w2cb):
    def body(si_ref, su_ref, di_ref, du_ref, b_ref, wc_ref, wcb_ref,
             oc_ref, ocb_ref):
        b = b_ref[...]
        di = jnp.maximum(di_ref[...], 1.0)
        du = jnp.maximum(du_ref[...], 1.0)
        hi = jnp.maximum(si_ref[...] / di + b, 0.0)
        hu = jnp.maximum(su_ref[...] / du + b, 0.0)
        oc_ref[...] = jnp.dot(hu, wc_ref[...], preferred_element_type=jnp.float32)
        ocb_ref[...] = jnp.dot(hi, wcb_ref[...], preferred_element_type=jnp.float32)

    return pl.pallas_call(
        body,
        grid=(N // _TCM,),
        in_specs=[pl.BlockSpec((_TCM, DF), lambda i: (i, 0)),
                  pl.BlockSpec((_TCM, DF), lambda i: (i, 0)),
                  pl.BlockSpec((_TCM, 1), lambda i: (i, 0)),
                  pl.BlockSpec((_TCM, 1), lambda i: (i, 0)),
                  pl.BlockSpec((1, DF), lambda i: (0, 0)),
                  pl.BlockSpec((DF, DF), lambda i: (0, 0)),
                  pl.BlockSpec((DF, DF), lambda i: (0, 0))],
        out_specs=[pl.BlockSpec((_TCM, DF), lambda i: (i, 0)),
                   pl.BlockSpec((_TCM, DF), lambda i: (i, 0))],
        out_shape=[jax.ShapeDtypeStruct((N, DF), jnp.float32),
                   jax.ShapeDtypeStruct((N, DF), jnp.float32)],
    )(s_i, s_u, deg_i, deg_u, b1, w2c, w2cb)


def _tc_fin(s2i, s2u, deg_i, deg_u, b2, wpu, wpi, bppad):
    def body(si_ref, su_ref, di_ref, du_ref, b_ref, wpu_ref, wpi_ref,
             bp_ref, oa_ref, ob_ref):
        b = b_ref[...]
        di = jnp.maximum(di_ref[...], 1.0)
        du = jnp.maximum(du_ref[...], 1.0)
        hi = si_ref[...] / di + b
        hu = su_ref[...] / du + b
        oa_ref[...] = jnp.dot(hu, wpu_ref[...],
                              preferred_element_type=jnp.float32)
        ob_ref[...] = jnp.dot(hi, wpi_ref[...],
                              preferred_element_type=jnp.float32) + bp_ref[...]

    return pl.pallas_call(
        body,
        grid=(N // _TCM,),
        in_specs=[pl.BlockSpec((_TCM, DF), lambda i: (i, 0)),
                  pl.BlockSpec((_TCM, DF), lambda i: (i, 0)),
                  pl.BlockSpec((_TCM, 1), lambda i: (i, 0)),
                  pl.BlockSpec((_TCM, 1), lambda i: (i, 0)),
                  pl.BlockSpec((1, DF), lambda i: (0, 0)),
                  pl.BlockSpec((DF, DF), lambda i: (0, 0)),
                  pl.BlockSpec((DF, DF), lambda i: (0, 0)),
                  pl.BlockSpec((1, DF), lambda i: (0, 0))],
        out_specs=[pl.BlockSpec((_TCM, DF), lambda i: (i, 0)),
                   pl.BlockSpec((_TCM, DF), lambda i: (i, 0))],
        out_shape=[jax.ShapeDtypeStruct((N, DF), jnp.float32),
                   jax.ShapeDtypeStruct((N, DF), jnp.float32)],
    )(s2i, s2u, deg_i, deg_u, b2, wpu, wpi, bppad)


def _prep_edges(edges):
    src = edges[0].reshape(NSUB, NCHUNK, CH)
    dst = edges[1].reshape(NSUB, NCHUNK, CH)
    return src, dst


def kernel(x_user, x_item, edge_clicks, edge_clicked_by, neg_edge_clicks,
           W1_clicks, W1_clicked_by, b1, W2_clicks, W2_clicked_by, b2, Wp, bp):
    src_c, dst_c = _prep_edges(edge_clicks)
    src_cb, dst_cb = _prep_edges(edge_clicked_by)
    zeros = jnp.zeros((NPAD, DF), jnp.float32)
    ones = jnp.ones((CH, DF), jnp.float32)

    m1c, m1cb = _tc_pre(x_user, x_item, W1_clicks, W1_clicked_by)
    dgi, dgu = _deg(dst_c, dst_cb, zeros, ones)
    s1i, s1u = _agg(m1c, m1cb, src_c, dst_c, src_cb, dst_cb, zeros)
    deg_i = dgi[:, :1]
    deg_u = dgu[:, :1]
    m2c, m2cb = _tc_mid(s1i, s1u, deg_i, deg_u, b1.reshape(1, DF),
                        W2_clicks, W2_clicked_by)
    s2i, s2u = _agg(m2c, m2cb, src_c, dst_c, src_cb, dst_cb, zeros)

    wpu = jnp.pad(Wp[:DF], ((0, 0), (0, DF - 2)))
    wpi = jnp.pad(Wp[DF:], ((0, 0), (0, DF - 2)))
    bppad = jnp.pad(bp, (0, DF - 2)).reshape(1, DF)
    qa, qb = _tc_fin(s2i, s2u, deg_i, deg_u, b2.reshape(1, DF),
                     wpu, wpi, bppad)

    pos_e = edge_clicks.reshape(2, NSUB, PNC, PCH)
    neg_e = neg_edge_clicks.reshape(2, NSUB, PNC, PCH)
    pos_f, neg_f = _pred(qa, qb, pos_e, neg_e)
    return pos_f[:, :2], neg_f[:, :2]
